# Initial kernel scaffold; baseline (speedup 1.0000x reference)
import functools

import jax
import jax.numpy as jnp
from jax import lax
from jax.experimental import pallas as pl
from jax.experimental.pallas import tpu as pltpu
from jax.experimental.pallas import tpu_sc as plsc

N, E, D, R, H = 10000, 320000, 128, 32, 64

NW = 32
EPT = E // NW
C = 80
NCHUNK = EPT // C
GRP = C // 16


def _pre_body(x_ref, rel_ref, u_ref,
              w1c_ref, b1c_ref, w2c_ref, b2c_ref,
              w1s_ref, b1s_ref, w2s_ref, b2s_ref,
              w1d_ref, b1d_ref, w2d_ref, b2d_ref,
              w1e_ref, b1e_ref, w2e_ref, b2e_ref,
              p_ref, q_ref, ssrc_ref, sdst_ref, sedge_ref, noise_ref):
    x = x_ref[...]
    rel = rel_ref[...]
    p_ref[...] = jnp.dot(x, w1c_ref[...], preferred_element_type=jnp.float32)
    q_ref[...] = (jnp.dot(rel, w1c_ref[...], preferred_element_type=jnp.float32)
                  + b1c_ref[...][None, :])

    def mlp(inp, w1, b1, w2, b2):
        hid = jnp.maximum(jnp.dot(inp, w1, preferred_element_type=jnp.float32)
                          + b1[None, :], 0.0)
        return jnp.dot(hid, w2, preferred_element_type=jnp.float32) + b2[None, :]

    ssrc_ref[...] = mlp(x, w1s_ref[...], b1s_ref[...], w2s_ref[...], b2s_ref[...])
    sdst_ref[...] = mlp(x, w1d_ref[...], b1d_ref[...], w2d_ref[...], b2d_ref[...])
    sedge_ref[...] = (mlp(rel, w1e_ref[...], b1e_ref[...], w2e_ref[...], b2e_ref[...])
                      + b2c_ref[...][None, :])

    bias = 0.0 + 0.0001
    u = u_ref[...]
    eps = (bias - (1.0 - bias)) * u + (1.0 - bias)
    noise_ref[...] = jnp.log(eps) - jnp.log(1.0 - eps)


_precompute = pl.pallas_call(
    _pre_body,
    out_shape=[
        jax.ShapeDtypeStruct((N, H), jnp.float32),
        jax.ShapeDtypeStruct((R, H), jnp.float32),
        jax.ShapeDtypeStruct((N, 1), jnp.float32),
        jax.ShapeDtypeStruct((N, 1), jnp.float32),
        jax.ShapeDtypeStruct((R, 1), jnp.float32),
        jax.ShapeDtypeStruct((E // 128, 128), jnp.float32),
    ],
)


def _sc_body(head_hbm, tail_hbm, rtype_hbm, noise_hbm,
             p_hbm, ssrc_hbm, sdst_hbm, q_hbm, sedge_hbm, w2_hbm,
             out_hbm,
             hidx, tidx, ridx, nz, hrows, trows, outv,
             ssrc, sdst, qv, sedge, w2v, sem):
    wid = lax.axis_index("s") * 2 + lax.axis_index("c")
    base = wid * EPT

    pltpu.sync_copy(ssrc_hbm, ssrc)
    pltpu.sync_copy(sdst_hbm, sdst)
    pltpu.sync_copy(q_hbm, qv)
    pltpu.sync_copy(sedge_hbm, sedge)
    pltpu.sync_copy(w2_hbm, w2v)

    lanes = lax.iota(jnp.int32, 16)

    def chunk_body(ci, carry):
        off = base + ci * C
        pltpu.sync_copy(head_hbm.at[pl.ds(off, C)], hidx)
        pltpu.sync_copy(tail_hbm.at[pl.ds(off, C)], tidx)
        pltpu.sync_copy(rtype_hbm.at[pl.ds(off, C)], ridx)
        pltpu.sync_copy(noise_hbm.at[pl.ds(off, C)], nz)
        cp1 = pltpu.async_copy(p_hbm.at[hidx], hrows, sem)
        cp2 = pltpu.async_copy(p_hbm.at[tidx], trows, sem)
        cp1.wait()
        cp2.wait()
        for g in range(GRP):
            sl = pl.ds(g * 16, 16)
            hv = hidx[sl]
            tv = tidx[sl]
            rv = ridx[sl]
            acc = (plsc.load_gather(ssrc, [hv])
                   + plsc.load_gather(sdst, [tv])
                   + plsc.load_gather(sedge, [rv])
                   + nz[sl])
            rows = lanes + g * 16
            for f in range(H):
                fv = jnp.full((16,), f, jnp.int32)
                ph = plsc.load_gather(hrows, [rows, fv])
                pt = plsc.load_gather(trows, [rows, fv])
                qf = plsc.load_gather(qv, [rv, fv])
                w2f = plsc.load_gather(w2v, [fv])
                acc = acc + jnp.maximum(ph + pt + qf, 0.0) * w2f
            gate = acc * 2.0
            z = jnp.exp(-jnp.abs(gate))
            sig = jnp.where(gate >= 0.0, 1.0 / (1.0 + z), z / (1.0 + z))
            outv[sl] = sig
        pltpu.sync_copy(outv, out_hbm.at[pl.ds(off, C)])
        return carry

    lax.fori_loop(0, NCHUNK, chunk_body, 0)


_sc_call = functools.partial(
    pl.kernel,
    mesh=plsc.VectorSubcoreMesh(core_axis_name="c", subcore_axis_name="s"),
    out_type=jax.ShapeDtypeStruct((E,), jnp.float32),
    scratch_types=[
        pltpu.VMEM((C,), jnp.int32),
        pltpu.VMEM((C,), jnp.int32),
        pltpu.VMEM((C,), jnp.int32),
        pltpu.VMEM((C,), jnp.float32),
        pltpu.VMEM((C, H), jnp.float32),
        pltpu.VMEM((C, H), jnp.float32),
        pltpu.VMEM((C,), jnp.float32),
        pltpu.VMEM((N,), jnp.float32),
        pltpu.VMEM((N,), jnp.float32),
        pltpu.VMEM((R, H), jnp.float32),
        pltpu.VMEM((R,), jnp.float32),
        pltpu.VMEM((H,), jnp.float32),
        pltpu.SemaphoreType.DMA,
    ],
)(_sc_body)


def kernel(edge_index, edge_type, all_embed, relation_emb,
           W1_con, b1_con, W2_con, b2_con,
           W1_src, b1_src, W2_src, b2_src,
           W1_dst, b1_dst, W2_dst, b2_dst,
           W1_edge, b1_edge, W2_edge, b2_edge):
    u = jax.random.uniform(jax.random.key(42), (E,), dtype=jnp.float32)
    p, q, ssrc, sdst, sedge, noise = _precompute(
        all_embed, relation_emb, u.reshape(E // 128, 128),
        W1_con, b1_con, W2_con, b2_con,
        W1_src, b1_src, W2_src, b2_src,
        W1_dst, b1_dst, W2_dst, b2_dst,
        W1_edge, b1_edge, W2_edge, b2_edge,
    )
    head = edge_index[0].astype(jnp.int32)
    tail = edge_index[1].astype(jnp.int32)
    rtype = edge_type.astype(jnp.int32)
    return _sc_call(
        head, tail, rtype, noise.reshape(E),
        p, ssrc.reshape(N), sdst.reshape(N), q, sedge.reshape(R),
        W2_con.reshape(H),
    )


# speedup vs baseline: 2.0744x; 2.0744x over previous
import functools

import jax
import jax.numpy as jnp
from jax import lax
from jax.experimental import pallas as pl
from jax.experimental.pallas import tpu as pltpu
from jax.experimental.pallas import tpu_sc as plsc

N, E, D, R, H = 10000, 320000, 128, 32, 64

NW = 32
EPT = E // NW
C = 80
NCHUNK = EPT // C
GRP = C // 16


def _pre_body(x_ref, rel_ref, u_ref,
              w1c_ref, b1c_ref, w2c_ref, b2c_ref,
              w1s_ref, b1s_ref, w2s_ref, b2s_ref,
              w1d_ref, b1d_ref, w2d_ref, b2d_ref,
              w1e_ref, b1e_ref, w2e_ref, b2e_ref,
              p_ref, q_ref, ssrc_ref, sdst_ref, sedge_ref, noise_ref, w2x_ref):
    x = x_ref[...]
    rel = rel_ref[...]

    def dot(a, b):
        return jnp.dot(a, b, preferred_element_type=jnp.float32,
                       precision=lax.Precision.HIGHEST)

    p_ref[...] = dot(x, w1c_ref[...])
    q_ref[...] = dot(rel, w1c_ref[...]) + b1c_ref[...][None, :]

    def mlp(inp, w1, b1, w2, b2):
        hid = jnp.maximum(dot(inp, w1) + b1[None, :], 0.0)
        return dot(hid, w2) + b2[None, :]

    ssrc_ref[...] = mlp(x, w1s_ref[...], b1s_ref[...], w2s_ref[...], b2s_ref[...])
    sdst_ref[...] = mlp(x, w1d_ref[...], b1d_ref[...], w2d_ref[...], b2d_ref[...])
    sedge_ref[...] = (mlp(rel, w1e_ref[...], b1e_ref[...], w2e_ref[...], b2e_ref[...])
                      + b2c_ref[...][None, :])

    bias = 0.0 + 0.0001
    u = u_ref[...]
    eps = (bias - (1.0 - bias)) * u + (1.0 - bias)
    noise_ref[...] = jnp.log(eps) - jnp.log(1.0 - eps)

    w2x_ref[...] = jnp.broadcast_to(w2c_ref[...].reshape(H, 1), (H, 16))


_precompute = pl.pallas_call(
    _pre_body,
    out_shape=[
        jax.ShapeDtypeStruct((N, H), jnp.float32),
        jax.ShapeDtypeStruct((R, H), jnp.float32),
        jax.ShapeDtypeStruct((N, 1), jnp.float32),
        jax.ShapeDtypeStruct((N, 1), jnp.float32),
        jax.ShapeDtypeStruct((R, 1), jnp.float32),
        jax.ShapeDtypeStruct((E // 128, 128), jnp.float32),
        jax.ShapeDtypeStruct((H, 16), jnp.float32),
    ],
)


def _sc_body(head_hbm, tail_hbm, rtype_hbm, noise_hbm,
             p_hbm, ssrc_hbm, sdst_hbm, q_hbm, sedge_hbm, w2_hbm,
             out_hbm,
             hidx, tidx, ridx, nz, hrows, trows, outv,
             ssrc, sdst, qv, sedge, w2v, sem):
    wid = lax.axis_index("s") * 2 + lax.axis_index("c")
    base = wid * EPT

    pltpu.sync_copy(ssrc_hbm, ssrc)
    pltpu.sync_copy(sdst_hbm, sdst)
    pltpu.sync_copy(q_hbm, qv)
    pltpu.sync_copy(sedge_hbm, sedge)
    pltpu.sync_copy(w2_hbm, w2v)

    lanes = lax.iota(jnp.int32, 16)

    def chunk_body(ci, carry):
        off = base + ci * C
        pltpu.sync_copy(head_hbm.at[pl.ds(off, C)], hidx)
        pltpu.sync_copy(tail_hbm.at[pl.ds(off, C)], tidx)
        pltpu.sync_copy(rtype_hbm.at[pl.ds(off, C)], ridx)
        pltpu.sync_copy(noise_hbm.at[pl.ds(off, C)], nz)
        cp1 = pltpu.async_copy(p_hbm.at[hidx], hrows, sem)
        cp2 = pltpu.async_copy(p_hbm.at[tidx], trows, sem)
        cp1.wait()
        cp2.wait()
        for g in range(GRP):
            sl = pl.ds(g * 16, 16)
            hv = hidx[sl]
            tv = tidx[sl]
            rv = ridx[sl]
            acc = (plsc.load_gather(ssrc, [hv])
                   + plsc.load_gather(sdst, [tv])
                   + plsc.load_gather(sedge, [rv])
                   + nz[sl])
            rows = lanes + g * 16
            for f in range(H):
                fv = jnp.full((16,), f, jnp.int32)
                ph = plsc.load_gather(hrows, [rows, fv])
                pt = plsc.load_gather(trows, [rows, fv])
                qf = plsc.load_gather(qv, [rv, fv])
                w2f = plsc.load_gather(w2v, [lanes + (f * 16)])
                acc = acc + jnp.maximum(ph + pt + qf, 0.0) * w2f
            gate = acc * 2.0
            z = jnp.exp(-jnp.abs(gate))
            sig = jnp.where(gate >= 0.0, 1.0 / (1.0 + z), z / (1.0 + z))
            outv[sl] = sig
        pltpu.sync_copy(outv, out_hbm.at[pl.ds(off, C)])
        return carry

    lax.fori_loop(0, NCHUNK, chunk_body, 0)


_sc_call = functools.partial(
    pl.kernel,
    mesh=plsc.VectorSubcoreMesh(core_axis_name="c", subcore_axis_name="s"),
    out_type=jax.ShapeDtypeStruct((E,), jnp.float32),
    compiler_params=pltpu.CompilerParams(
        needs_layout_passes=False, use_tc_tiling_on_sc=False),
    scratch_types=[
        pltpu.VMEM((C,), jnp.int32),
        pltpu.VMEM((C,), jnp.int32),
        pltpu.VMEM((C,), jnp.int32),
        pltpu.VMEM((C,), jnp.float32),
        pltpu.VMEM((C, H), jnp.float32),
        pltpu.VMEM((C, H), jnp.float32),
        pltpu.VMEM((C,), jnp.float32),
        pltpu.VMEM((N,), jnp.float32),
        pltpu.VMEM((N,), jnp.float32),
        pltpu.VMEM((R, H), jnp.float32),
        pltpu.VMEM((R,), jnp.float32),
        pltpu.VMEM((H * 16,), jnp.float32),
        pltpu.SemaphoreType.DMA,
    ],
)(_sc_body)


def kernel(edge_index, edge_type, all_embed, relation_emb,
           W1_con, b1_con, W2_con, b2_con,
           W1_src, b1_src, W2_src, b2_src,
           W1_dst, b1_dst, W2_dst, b2_dst,
           W1_edge, b1_edge, W2_edge, b2_edge):
    u = jax.random.uniform(jax.random.key(42), (E,), dtype=jnp.float32)
    p, q, ssrc, sdst, sedge, noise, w2x = _precompute(
        all_embed, relation_emb, u.reshape(E // 128, 128),
        W1_con, b1_con, W2_con, b2_con,
        W1_src, b1_src, W2_src, b2_src,
        W1_dst, b1_dst, W2_dst, b2_dst,
        W1_edge, b1_edge, W2_edge, b2_edge,
    )
    head = edge_index[0].astype(jnp.int32)
    tail = edge_index[1].astype(jnp.int32)
    rtype = edge_type.astype(jnp.int32)
    return _sc_call(
        head, tail, rtype, noise.reshape(E),
        p, ssrc.reshape(N), sdst.reshape(N), q, sedge.reshape(R),
        w2x.reshape(H * 16),
    )


# speedup vs baseline: 2.6420x; 1.2737x over previous
import functools

import jax
import jax.numpy as jnp
from jax import lax
from jax.experimental import pallas as pl
from jax.experimental.pallas import tpu as pltpu
from jax.experimental.pallas import tpu_sc as plsc

N, E, D, R, H = 10000, 320000, 128, 32, 64

NW = 32
EPT = E // NW
SUB = 80
NSUB = EPT // SUB
NBUF = 5
LOOKAHEAD = 4
GRP = SUB // 16


def _pre_body(x_ref, rel_ref,
              w1c_ref, b1c_ref, w2c_ref, b2c_ref,
              w1s_ref, b1s_ref, w2s_ref, b2s_ref,
              w1d_ref, b1d_ref, w2d_ref, b2d_ref,
              w1e_ref, b1e_ref, w2e_ref, b2e_ref,
              p_ref, q_ref, ssrc_ref, sdst_ref, sedge_ref, w2x_ref):
    x = x_ref[...]
    rel = rel_ref[...]

    def dot(a, b):
        return jnp.dot(a, b, preferred_element_type=jnp.float32,
                       precision=lax.Precision.HIGHEST)

    p_ref[...] = dot(x, w1c_ref[...])
    q_ref[...] = dot(rel, w1c_ref[...]) + b1c_ref[...][None, :]

    def mlp(inp, w1, b1, w2, b2):
        hid = jnp.maximum(dot(inp, w1) + b1[None, :], 0.0)
        return dot(hid, w2) + b2[None, :]

    ssrc_ref[...] = mlp(x, w1s_ref[...], b1s_ref[...], w2s_ref[...], b2s_ref[...])
    sdst_ref[...] = mlp(x, w1d_ref[...], b1d_ref[...], w2d_ref[...], b2d_ref[...])
    sedge_ref[...] = (mlp(rel, w1e_ref[...], b1e_ref[...], w2e_ref[...], b2e_ref[...])
                      + b2c_ref[...][None, :])

    w2x_ref[...] = jnp.broadcast_to(w2c_ref[...].reshape(H, 1), (H, 16))


_precompute = pl.pallas_call(
    _pre_body,
    out_shape=[
        jax.ShapeDtypeStruct((N, H), jnp.float32),
        jax.ShapeDtypeStruct((R, H), jnp.float32),
        jax.ShapeDtypeStruct((N, 1), jnp.float32),
        jax.ShapeDtypeStruct((N, 1), jnp.float32),
        jax.ShapeDtypeStruct((R, 1), jnp.float32),
        jax.ShapeDtypeStruct((H, 16), jnp.float32),
    ],
)


def _epi_body(w_ref, u_ref, out_ref):
    bias = 0.0 + 0.0001
    u = u_ref[...]
    eps = (bias - (1.0 - bias)) * u + (1.0 - bias)
    noise = jnp.log(eps) - jnp.log(1.0 - eps)
    out_ref[...] = jax.nn.sigmoid((noise + w_ref[...]) * 2.0)


_epilogue = pl.pallas_call(
    _epi_body,
    out_shape=jax.ShapeDtypeStruct((E // 128, 128), jnp.float32),
)


def _sc_body(head_hbm, tail_hbm, rtype_hbm,
             p_hbm, ssrc_hbm, sdst_hbm, q_hbm, sedge_hbm, w2_hbm,
             out_hbm,
             hidx, tidx, ridx, outv,
             ssrc, sdst, qv, sedge, w2v,
             hrows0, hrows1, hrows2, hrows3, hrows4,
             trows0, trows1, trows2, trows3, trows4,
             semh0, semh1, semh2, semh3, semh4,
             semt0, semt1, semt2, semt3, semt4):
    wid = lax.axis_index("s") * 2 + lax.axis_index("c")
    rbase = wid * NSUB

    hrows = [hrows0, hrows1, hrows2, hrows3, hrows4]
    trows = [trows0, trows1, trows2, trows3, trows4]
    semh = [semh0, semh1, semh2, semh3, semh4]
    semt = [semt0, semt1, semt2, semt3, semt4]

    pltpu.sync_copy(head_hbm.at[pl.ds(rbase, NSUB), :], hidx)
    pltpu.sync_copy(tail_hbm.at[pl.ds(rbase, NSUB), :], tidx)
    pltpu.sync_copy(rtype_hbm.at[pl.ds(rbase, NSUB), :], ridx)
    pltpu.sync_copy(ssrc_hbm, ssrc)
    pltpu.sync_copy(sdst_hbm, sdst)
    pltpu.sync_copy(q_hbm, qv)
    pltpu.sync_copy(sedge_hbm, sedge)
    pltpu.sync_copy(w2_hbm, w2v)

    lanes = lax.iota(jnp.int32, 16)

    def fire(s, slot):
        pltpu.async_copy(p_hbm.at[hidx.at[s]], hrows[slot], semh[slot])
        pltpu.async_copy(p_hbm.at[tidx.at[s]], trows[slot], semt[slot])

    def drain(slot):
        pltpu.make_async_copy(p_hbm.at[hidx.at[0]], hrows[slot], semh[slot]).wait()
        pltpu.make_async_copy(p_hbm.at[tidx.at[0]], trows[slot], semt[slot]).wait()

    for s in range(LOOKAHEAD):
        fire(s, s)

    def outer(so, carry):
        for j in range(NBUF):
            s = so * NBUF + j
            drain(j)
            hr = hrows[j]
            tr = trows[j]
            hvs = []
            tvs = []
            rvs = []
            accs = []
            for g in range(GRP):
                gsl = pl.ds(g * 16, 16)
                hv = hidx[s, gsl]
                tv = tidx[s, gsl]
                rv = ridx[s, gsl]
                hvs.append(hv)
                tvs.append(tv)
                rvs.append(rv)
                accs.append(plsc.load_gather(ssrc, [hv])
                            + plsc.load_gather(sdst, [tv])
                            + plsc.load_gather(sedge, [rv]))

            def feat(f, accs):
                fv = jnp.full((16,), f, jnp.int32)
                w2f = plsc.load_gather(w2v, [lanes + f * 16])
                out = []
                for g in range(GRP):
                    rows = lanes + g * 16
                    ph = plsc.load_gather(hr, [rows, fv])
                    pt = plsc.load_gather(tr, [rows, fv])
                    qf = plsc.load_gather(qv, [rvs[g], fv])
                    out.append(accs[g]
                               + jnp.maximum(ph + pt + qf, 0.0) * w2f)
                return tuple(out)

            accs = lax.fori_loop(0, H, feat, tuple(accs), unroll=4)
            for g in range(GRP):
                outv[pl.ds(s * SUB + g * 16, 16)] = accs[g]

            @pl.when(s + LOOKAHEAD < NSUB)
            def _():
                fire(s + LOOKAHEAD, (j + LOOKAHEAD) % NBUF)
        return carry

    lax.fori_loop(0, NSUB // NBUF, outer, 0)
    pltpu.sync_copy(outv, out_hbm.at[pl.ds(wid * EPT, EPT)])


_sc_call = functools.partial(
    pl.kernel,
    mesh=plsc.VectorSubcoreMesh(core_axis_name="c", subcore_axis_name="s"),
    out_type=jax.ShapeDtypeStruct((E,), jnp.float32),
    scratch_types=(
        [pltpu.VMEM((NSUB, SUB), jnp.int32)] * 3
        + [pltpu.VMEM((EPT,), jnp.float32)]
        + [pltpu.VMEM((N,), jnp.float32)] * 2
        + [pltpu.VMEM((R, H), jnp.float32),
           pltpu.VMEM((R,), jnp.float32),
           pltpu.VMEM((H * 16,), jnp.float32)]
        + [pltpu.VMEM((SUB, H), jnp.float32)] * (2 * NBUF)
        + [pltpu.SemaphoreType.DMA] * (2 * NBUF)
    ),
    compiler_params=pltpu.CompilerParams(
        needs_layout_passes=False, use_tc_tiling_on_sc=False),
)(_sc_body)


def kernel(edge_index, edge_type, all_embed, relation_emb,
           W1_con, b1_con, W2_con, b2_con,
           W1_src, b1_src, W2_src, b2_src,
           W1_dst, b1_dst, W2_dst, b2_dst,
           W1_edge, b1_edge, W2_edge, b2_edge):
    p, q, ssrc, sdst, sedge, w2x = _precompute(
        all_embed, relation_emb,
        W1_con, b1_con, W2_con, b2_con,
        W1_src, b1_src, W2_src, b2_src,
        W1_dst, b1_dst, W2_dst, b2_dst,
        W1_edge, b1_edge, W2_edge, b2_edge,
    )
    head = edge_index[0].astype(jnp.int32).reshape(E // SUB, SUB)
    tail = edge_index[1].astype(jnp.int32).reshape(E // SUB, SUB)
    rtype = edge_type.astype(jnp.int32).reshape(E // SUB, SUB)
    w = _sc_call(
        head, tail, rtype,
        p, ssrc.reshape(N), sdst.reshape(N), q, sedge.reshape(R),
        w2x.reshape(H * 16),
    )
    u = jax.random.uniform(jax.random.key(42), (E,), dtype=jnp.float32)
    out = _epilogue(w.reshape(E // 128, 128), u.reshape(E // 128, 128))
    return out.reshape(E)


# speedup vs baseline: 2.6970x; 1.0208x over previous
import functools

import jax
import jax.numpy as jnp
from jax import lax
from jax.experimental import pallas as pl
from jax.experimental.pallas import tpu as pltpu
from jax.experimental.pallas import tpu_sc as plsc

N, E, D, R, H = 10000, 320000, 128, 32, 64

NW = 32
EPT = E // NW
SUB = 16
NSUB = EPT // SUB
NBUF = 5
LOOKAHEAD = 4


def _pre_body(x_ref, rel_ref,
              w1c_ref, b1c_ref, w2c_ref, b2c_ref,
              w1s_ref, b1s_ref, w2s_ref, b2s_ref,
              w1d_ref, b1d_ref, w2d_ref, b2d_ref,
              w1e_ref, b1e_ref, w2e_ref, b2e_ref,
              p_ref, q_ref, ssrc_ref, sdst_ref, sedge_ref, w2x_ref):
    x = x_ref[...]
    rel = rel_ref[...]

    def dot(a, b):
        return jnp.dot(a, b, preferred_element_type=jnp.float32,
                       precision=lax.Precision.HIGHEST)

    p_ref[...] = dot(x, w1c_ref[...])
    q_ref[...] = dot(rel, w1c_ref[...]) + b1c_ref[...][None, :]

    def mlp(inp, w1, b1, w2, b2):
        hid = jnp.maximum(dot(inp, w1) + b1[None, :], 0.0)
        return dot(hid, w2) + b2[None, :]

    ssrc_ref[...] = mlp(x, w1s_ref[...], b1s_ref[...], w2s_ref[...], b2s_ref[...])
    sdst_ref[...] = mlp(x, w1d_ref[...], b1d_ref[...], w2d_ref[...], b2d_ref[...])
    sedge_ref[...] = (mlp(rel, w1e_ref[...], b1e_ref[...], w2e_ref[...], b2e_ref[...])
                      + b2c_ref[...][None, :])

    w2x_ref[...] = jnp.broadcast_to(w2c_ref[...].reshape(H, 1), (H, 16))


_precompute = pl.pallas_call(
    _pre_body,
    out_shape=[
        jax.ShapeDtypeStruct((N, H), jnp.float32),
        jax.ShapeDtypeStruct((R, H), jnp.float32),
        jax.ShapeDtypeStruct((N, 1), jnp.float32),
        jax.ShapeDtypeStruct((N, 1), jnp.float32),
        jax.ShapeDtypeStruct((R, 1), jnp.float32),
        jax.ShapeDtypeStruct((H, 16), jnp.float32),
    ],
)


def _epi_body(w_ref, u_ref, out_ref):
    bias = 0.0 + 0.0001
    u = u_ref[...]
    eps = (bias - (1.0 - bias)) * u + (1.0 - bias)
    noise = jnp.log(eps) - jnp.log(1.0 - eps)
    out_ref[...] = jax.nn.sigmoid((noise + w_ref[...]) * 2.0)


_epilogue = pl.pallas_call(
    _epi_body,
    out_shape=jax.ShapeDtypeStruct((E // 128, 128), jnp.float32),
)


def _sc_body(head_hbm, tail_hbm, rtype_hbm,
             p_hbm, ssrc_hbm, sdst_hbm, q_hbm, sedge_hbm, w2_hbm,
             out_hbm,
             hidx, tidx, ridx, outv,
             ssrc, sdst, qv, sedge, w2v,
             hrows0, hrows1, hrows2, hrows3, hrows4,
             trows0, trows1, trows2, trows3, trows4,
             p_sh,
             semh0, semh1, semh2, semh3, semh4,
             semt0, semt1, semt2, semt3, semt4):
    wid = lax.axis_index("s") * 2 + lax.axis_index("c")
    rbase = wid * NSUB

    hrows = [hrows0, hrows1, hrows2, hrows3, hrows4]
    trows = [trows0, trows1, trows2, trows3, trows4]
    semh = [semh0, semh1, semh2, semh3, semh4]
    semt = [semt0, semt1, semt2, semt3, semt4]

    @pl.when(lax.axis_index("s") == 0)
    def _():
        pltpu.sync_copy(p_hbm, p_sh)

    pltpu.sync_copy(head_hbm.at[pl.ds(rbase, NSUB), :], hidx)
    pltpu.sync_copy(tail_hbm.at[pl.ds(rbase, NSUB), :], tidx)
    pltpu.sync_copy(rtype_hbm.at[pl.ds(rbase, NSUB), :], ridx)
    pltpu.sync_copy(ssrc_hbm, ssrc)
    pltpu.sync_copy(sdst_hbm, sdst)
    pltpu.sync_copy(q_hbm, qv)
    pltpu.sync_copy(sedge_hbm, sedge)
    pltpu.sync_copy(w2_hbm, w2v)
    plsc.subcore_barrier()

    lanes = lax.iota(jnp.int32, 16)

    def fire(s, slot):
        pltpu.async_copy(p_sh.at[hidx.at[s]], hrows[slot], semh[slot])
        pltpu.async_copy(p_sh.at[tidx.at[s]], trows[slot], semt[slot])

    def drain(slot):
        pltpu.make_async_copy(p_sh.at[hidx.at[0]], hrows[slot], semh[slot]).wait()
        pltpu.make_async_copy(p_sh.at[tidx.at[0]], trows[slot], semt[slot]).wait()

    for s in range(LOOKAHEAD):
        fire(s, s)

    def outer(so, carry):
        for j in range(NBUF):
            s = so * NBUF + j
            drain(j)
            hr = hrows[j]
            tr = trows[j]
            hv = hidx[s, :]
            tv = tidx[s, :]
            rv = ridx[s, :]
            acc0 = (plsc.load_gather(ssrc, [hv])
                    + plsc.load_gather(sdst, [tv])
                    + plsc.load_gather(sedge, [rv]))

            def feat(f, acc):
                fv = jnp.full((16,), f, jnp.int32)
                w2f = plsc.load_gather(w2v, [lanes + f * 16])
                ph = plsc.load_gather(hr, [lanes, fv])
                pt = plsc.load_gather(tr, [lanes, fv])
                qf = plsc.load_gather(qv, [rv, fv])
                return acc + jnp.maximum(ph + pt + qf, 0.0) * w2f

            acc = lax.fori_loop(0, H, feat, acc0, unroll=8)
            outv[pl.ds(s * SUB, 16)] = acc

            @pl.when(s + LOOKAHEAD < NSUB)
            def _():
                fire(s + LOOKAHEAD, (j + LOOKAHEAD) % NBUF)
        return carry

    lax.fori_loop(0, NSUB // NBUF, outer, 0)
    pltpu.sync_copy(outv, out_hbm.at[pl.ds(wid * EPT, EPT)])


_sc_call = functools.partial(
    pl.kernel,
    mesh=plsc.VectorSubcoreMesh(core_axis_name="c", subcore_axis_name="s"),
    out_type=jax.ShapeDtypeStruct((E,), jnp.float32),
    scratch_types=(
        [pltpu.VMEM((NSUB, SUB), jnp.int32)] * 3
        + [pltpu.VMEM((EPT,), jnp.float32)]
        + [pltpu.VMEM((N,), jnp.float32)] * 2
        + [pltpu.VMEM((R, H), jnp.float32),
           pltpu.VMEM((R,), jnp.float32),
           pltpu.VMEM((H * 16,), jnp.float32)]
        + [pltpu.VMEM((SUB, H), jnp.float32)] * (2 * NBUF)
        + [pltpu.VMEM_SHARED((N, H), jnp.float32)]
        + [pltpu.SemaphoreType.DMA] * (2 * NBUF)
    ),
    compiler_params=pltpu.CompilerParams(
        needs_layout_passes=False, use_tc_tiling_on_sc=False),
)(_sc_body)


def kernel(edge_index, edge_type, all_embed, relation_emb,
           W1_con, b1_con, W2_con, b2_con,
           W1_src, b1_src, W2_src, b2_src,
           W1_dst, b1_dst, W2_dst, b2_dst,
           W1_edge, b1_edge, W2_edge, b2_edge):
    p, q, ssrc, sdst, sedge, w2x = _precompute(
        all_embed, relation_emb,
        W1_con, b1_con, W2_con, b2_con,
        W1_src, b1_src, W2_src, b2_src,
        W1_dst, b1_dst, W2_dst, b2_dst,
        W1_edge, b1_edge, W2_edge, b2_edge,
    )
    head = edge_index[0].astype(jnp.int32).reshape(E // SUB, SUB)
    tail = edge_index[1].astype(jnp.int32).reshape(E // SUB, SUB)
    rtype = edge_type.astype(jnp.int32).reshape(E // SUB, SUB)
    w = _sc_call(
        head, tail, rtype,
        p, ssrc.reshape(N), sdst.reshape(N), q, sedge.reshape(R),
        w2x.reshape(H * 16),
    )
    u = jax.random.uniform(jax.random.key(42), (E,), dtype=jnp.float32)
    out = _epilogue(w.reshape(E // 128, 128), u.reshape(E // 128, 128))
    return out.reshape(E)


# speedup vs baseline: 5.8349x; 2.1635x over previous
import functools

import jax
import jax.numpy as jnp
from jax import lax
from jax.experimental import pallas as pl
from jax.experimental.pallas import tpu as pltpu
from jax.experimental.pallas import tpu_sc as plsc

N, E, D, R, H = 10000, 320000, 128, 32, 64

NSC = 2
NT = 16
FPT = H // NT
EPC = E // NSC
BLK = 4000
NBLK = EPC // BLK
GPB = BLK // 16
PRB = BLK // 32


def _pre_body(x_ref, rel_ref,
              w1c_ref, b1c_ref, w2c_ref, b2c_ref,
              w1s_ref, b1s_ref, w2s_ref, b2s_ref,
              w1d_ref, b1d_ref, w2d_ref, b2d_ref,
              w1e_ref, b1e_ref, w2e_ref, b2e_ref,
              p_ref, q_ref, ssrc_ref, sdst_ref, sedge_ref, w2x_ref):
    x = x_ref[...]
    rel = rel_ref[...]

    def dot(a, b):
        return jnp.dot(a, b, preferred_element_type=jnp.float32,
                       precision=lax.Precision.HIGHEST)

    p = dot(x, w1c_ref[...])
    q = dot(rel, w1c_ref[...]) + b1c_ref[...][None, :]
    p_ref[...] = p.T
    q_ref[...] = q.T

    def mlp(inp, w1, b1, w2, b2):
        hid = jnp.maximum(dot(inp, w1) + b1[None, :], 0.0)
        return dot(hid, w2) + b2[None, :]

    ssrc_ref[...] = mlp(x, w1s_ref[...], b1s_ref[...], w2s_ref[...], b2s_ref[...])
    sdst_ref[...] = mlp(x, w1d_ref[...], b1d_ref[...], w2d_ref[...], b2d_ref[...])
    sedge_ref[...] = (mlp(rel, w1e_ref[...], b1e_ref[...], w2e_ref[...], b2e_ref[...])
                      + b2c_ref[...][None, :])

    w2x_ref[...] = jnp.broadcast_to(w2c_ref[...].reshape(H, 1), (H, 16))


_precompute = pl.pallas_call(
    _pre_body,
    out_shape=[
        jax.ShapeDtypeStruct((H, N), jnp.float32),
        jax.ShapeDtypeStruct((H, R), jnp.float32),
        jax.ShapeDtypeStruct((N, 1), jnp.float32),
        jax.ShapeDtypeStruct((N, 1), jnp.float32),
        jax.ShapeDtypeStruct((R, 1), jnp.float32),
        jax.ShapeDtypeStruct((H, 16), jnp.float32),
    ],
)


def _epi_body(w_ref, u_ref, out_ref):
    bias = 0.0 + 0.0001
    u = u_ref[...]
    eps = (bias - (1.0 - bias)) * u + (1.0 - bias)
    noise = jnp.log(eps) - jnp.log(1.0 - eps)
    out_ref[...] = jax.nn.sigmoid((noise + w_ref[...]) * 2.0)


_epilogue = pl.pallas_call(
    _epi_body,
    out_shape=jax.ShapeDtypeStruct((E // 128, 128), jnp.float32),
)


def _sc_body(head_hbm, tail_hbm, rtype_hbm, rows_hbm,
             p_hbm, ssrc_hbm, sdst_hbm, q_hbm, sedge_hbm, w2_hbm,
             out_hbm,
             hb, tb, rb, partial, pslice, qslice,
             ssrc, sdst, sedge, w2v, rowidx,
             accum):
    sid = lax.axis_index("s")
    cid = lax.axis_index("c")
    k4 = sid * FPT

    pltpu.sync_copy(p_hbm.at[pl.ds(k4, FPT), :], pslice)
    pltpu.sync_copy(q_hbm.at[pl.ds(k4, FPT), :], qslice)
    pltpu.sync_copy(w2_hbm, w2v)
    pltpu.sync_copy(rows_hbm, rowidx)

    @pl.when(sid == 0)
    def _():
        pltpu.sync_copy(ssrc_hbm, ssrc)
        pltpu.sync_copy(sdst_hbm, sdst)
        pltpu.sync_copy(sedge_hbm, sedge)

    lanes = lax.iota(jnp.int32, 16)
    w2f = [plsc.load_gather(w2v, [lanes + (k4 + j) * 16]) for j in range(FPT)]

    def make_group(with_scalars):
        def group(gp, carry):
            for half in range(2):
                hv = hb[2 * gp + half, :]
                tv = tb[2 * gp + half, :]
                rv = rb[2 * gp + half, :]
                if with_scalars:
                    acc = (plsc.load_gather(ssrc, [hv])
                           + plsc.load_gather(sdst, [tv])
                           + plsc.load_gather(sedge, [rv]))
                else:
                    acc = jnp.zeros((16,), jnp.float32)
                for j in range(FPT):
                    fv = jnp.full((16,), j, jnp.int32)
                    ph = plsc.load_gather(pslice, [fv, hv])
                    pt = plsc.load_gather(pslice, [fv, tv])
                    qf = plsc.load_gather(qslice, [fv, rv])
                    acc = acc + jnp.maximum(ph + pt + qf, 0.0) * w2f[j]
                partial[gp, pl.ds(half * 16, 16)] = acc
            return carry
        return group

    def block(b, carry):
        rbase = cid * (EPC // 16) + b * GPB
        pltpu.sync_copy(head_hbm.at[pl.ds(rbase, GPB), :], hb)
        pltpu.sync_copy(tail_hbm.at[pl.ds(rbase, GPB), :], tb)
        pltpu.sync_copy(rtype_hbm.at[pl.ds(rbase, GPB), :], rb)

        _ = lax.cond(sid == 0,
                     lambda: lax.fori_loop(0, PRB, make_group(True), 0),
                     lambda: lax.fori_loop(0, PRB, make_group(False), 0))

        @pl.when(sid == 0)
        def _():
            pltpu.sync_copy(partial, accum)
        plsc.subcore_barrier()
        @pl.when(sid != 0)
        def _():
            pltpu.sync_copy(partial, accum.at[rowidx], add=True)
        plsc.subcore_barrier()
        @pl.when(sid == NT - 1)
        def _():
            obase = cid * (EPC // 32) + b * PRB
            pltpu.sync_copy(accum, out_hbm.at[pl.ds(obase, PRB), :])
        plsc.subcore_barrier()
        return carry

    lax.fori_loop(0, NBLK, block, 0)


_sc_call = functools.partial(
    pl.kernel,
    mesh=plsc.VectorSubcoreMesh(core_axis_name="c", subcore_axis_name="s"),
    out_type=jax.ShapeDtypeStruct((E // 32, 32), jnp.float32),
    scratch_types=(
        [pltpu.VMEM((GPB, 16), jnp.int32)] * 3
        + [pltpu.VMEM((PRB, 32), jnp.float32),
           pltpu.VMEM((FPT, N), jnp.float32),
           pltpu.VMEM((FPT, R), jnp.float32),
           pltpu.VMEM((N,), jnp.float32),
           pltpu.VMEM((N,), jnp.float32),
           pltpu.VMEM((R,), jnp.float32),
           pltpu.VMEM((H * 16,), jnp.float32),
           pltpu.VMEM((PRB,), jnp.int32),
           pltpu.VMEM_SHARED((PRB, 32), jnp.float32)]
    ),
    compiler_params=pltpu.CompilerParams(
        needs_layout_passes=False, use_tc_tiling_on_sc=False),
)(_sc_body)


def kernel(edge_index, edge_type, all_embed, relation_emb,
           W1_con, b1_con, W2_con, b2_con,
           W1_src, b1_src, W2_src, b2_src,
           W1_dst, b1_dst, W2_dst, b2_dst,
           W1_edge, b1_edge, W2_edge, b2_edge):
    p, q, ssrc, sdst, sedge, w2x = _precompute(
        all_embed, relation_emb,
        W1_con, b1_con, W2_con, b2_con,
        W1_src, b1_src, W2_src, b2_src,
        W1_dst, b1_dst, W2_dst, b2_dst,
        W1_edge, b1_edge, W2_edge, b2_edge,
    )
    head = edge_index[0].astype(jnp.int32).reshape(E // 16, 16)
    tail = edge_index[1].astype(jnp.int32).reshape(E // 16, 16)
    rtype = edge_type.astype(jnp.int32).reshape(E // 16, 16)
    rows = jnp.arange(PRB, dtype=jnp.int32)
    w = _sc_call(
        head, tail, rtype, rows,
        p, ssrc.reshape(N), sdst.reshape(N), q, sedge.reshape(R),
        w2x.reshape(H * 16),
    )
    u = jax.random.uniform(jax.random.key(42), (E,), dtype=jnp.float32)
    out = _epilogue(w.reshape(E // 128, 128), u.reshape(E // 128, 128))
    return out.reshape(E)


# speedup vs baseline: 6.8040x; 1.1661x over previous
import functools

import jax
import jax.numpy as jnp
from jax import lax
from jax.experimental import pallas as pl
from jax.experimental.pallas import tpu as pltpu
from jax.experimental.pallas import tpu_sc as plsc

N, E, D, R, H = 10000, 320000, 128, 32, 64

NSC = 2
NT = 16
FPT = H // NT
EPC = E // NSC
BLK = 8000
NBLK = EPC // BLK
GPB = BLK // 16
PRB = BLK // 32
RCH = 125


def _pre_body(x_ref, rel_ref,
              w1c_ref, b1c_ref, w2c_ref, b2c_ref,
              w1s_ref, b1s_ref, w2s_ref, b2s_ref,
              w1d_ref, b1d_ref, w2d_ref, b2d_ref,
              w1e_ref, b1e_ref, w2e_ref, b2e_ref,
              p_ref, q_ref, ssrc_ref, sdst_ref, sedge_ref, w2x_ref):
    x = x_ref[...]
    rel = rel_ref[...]

    def dot(a, b):
        return jnp.dot(a, b, preferred_element_type=jnp.float32,
                       precision=lax.Precision.HIGHEST)

    p = dot(x, w1c_ref[...])
    q = dot(rel, w1c_ref[...]) + b1c_ref[...][None, :]
    p_ref[...] = p.T
    q_ref[...] = q.T

    def mlp(inp, w1, b1, w2, b2):
        hid = jnp.maximum(dot(inp, w1) + b1[None, :], 0.0)
        return dot(hid, w2) + b2[None, :]

    ssrc_ref[...] = mlp(x, w1s_ref[...], b1s_ref[...], w2s_ref[...], b2s_ref[...])
    sdst_ref[...] = mlp(x, w1d_ref[...], b1d_ref[...], w2d_ref[...], b2d_ref[...])
    sedge_ref[...] = (mlp(rel, w1e_ref[...], b1e_ref[...], w2e_ref[...], b2e_ref[...])
                      + b2c_ref[...][None, :])

    w2x_ref[...] = jnp.broadcast_to(w2c_ref[...].reshape(H, 1), (H, 16))


_precompute = pl.pallas_call(
    _pre_body,
    out_shape=[
        jax.ShapeDtypeStruct((H, N), jnp.float32),
        jax.ShapeDtypeStruct((H, R), jnp.float32),
        jax.ShapeDtypeStruct((N, 1), jnp.float32),
        jax.ShapeDtypeStruct((N, 1), jnp.float32),
        jax.ShapeDtypeStruct((R, 1), jnp.float32),
        jax.ShapeDtypeStruct((H, 16), jnp.float32),
    ],
)


def _epi_body(w_ref, u_ref, out_ref):
    bias = 0.0 + 0.0001
    u = u_ref[...]
    eps = (bias - (1.0 - bias)) * u + (1.0 - bias)
    noise = jnp.log(eps) - jnp.log(1.0 - eps)
    out_ref[...] = jax.nn.sigmoid((noise + w_ref[...]) * 2.0)


_epilogue = pl.pallas_call(
    _epi_body,
    out_shape=jax.ShapeDtypeStruct((E // 128, 128), jnp.float32),
)


def _sc_body(head_hbm, tail_hbm, rtype_hbm, rows_hbm,
             p_hbm, ssrc_hbm, sdst_hbm, q_hbm, sedge_hbm, w2_hbm,
             out_hbm,
             hb, tb, rb, partial, pslice, qslice,
             ssrc, sdst, sedge, w2v, rowidx,
             accum):
    sid = lax.axis_index("s")
    cid = lax.axis_index("c")
    k4 = sid * FPT

    pltpu.sync_copy(p_hbm.at[pl.ds(k4, FPT), :], pslice)
    pltpu.sync_copy(q_hbm.at[pl.ds(k4, FPT), :], qslice)
    pltpu.sync_copy(w2_hbm, w2v)
    pltpu.sync_copy(rows_hbm, rowidx)

    @pl.when(sid == 0)
    def _():
        pltpu.sync_copy(ssrc_hbm, ssrc)
        pltpu.sync_copy(sdst_hbm, sdst)
        pltpu.sync_copy(sedge_hbm, sedge)

    lanes = lax.iota(jnp.int32, 16)
    w2f = [plsc.load_gather(w2v, [lanes + (k4 + j) * 16]) for j in range(FPT)]

    def make_group(with_scalars):
        def group(gp, carry):
            for half in range(2):
                hv = hb[2 * gp + half, :]
                tv = tb[2 * gp + half, :]
                rv = rb[2 * gp + half, :]
                if with_scalars:
                    acc = (plsc.load_gather(ssrc, [hv])
                           + plsc.load_gather(sdst, [tv])
                           + plsc.load_gather(sedge, [rv]))
                else:
                    acc = jnp.zeros((16,), jnp.float32)
                for j in range(FPT):
                    fv = jnp.full((16,), j, jnp.int32)
                    ph = plsc.load_gather(pslice, [fv, hv])
                    pt = plsc.load_gather(pslice, [fv, tv])
                    qf = plsc.load_gather(qslice, [fv, rv])
                    acc = acc + jnp.maximum(ph + pt + qf, 0.0) * w2f[j]
                partial[gp, pl.ds(half * 16, 16)] = acc
            return carry
        return group

    def block(b, carry):
        rbase = cid * (EPC // 16) + b * GPB
        pltpu.sync_copy(head_hbm.at[pl.ds(rbase, GPB), :], hb)
        pltpu.sync_copy(tail_hbm.at[pl.ds(rbase, GPB), :], tb)
        pltpu.sync_copy(rtype_hbm.at[pl.ds(rbase, GPB), :], rb)

        _ = lax.cond(sid == 0,
                     lambda: lax.fori_loop(0, PRB, make_group(True), 0),
                     lambda: lax.fori_loop(0, PRB, make_group(False), 0))

        @pl.when(sid == 0)
        def _():
            pltpu.sync_copy(partial, accum)
        plsc.subcore_barrier()
        @pl.when(sid != 0)
        def _():
            for j in range(PRB // RCH):
                pltpu.sync_copy(partial.at[pl.ds(j * RCH, RCH), :],
                                accum.at[rowidx.at[j]], add=True)
        plsc.subcore_barrier()
        @pl.when(sid == NT - 1)
        def _():
            obase = cid * (EPC // 32) + b * PRB
            pltpu.sync_copy(accum, out_hbm.at[pl.ds(obase, PRB), :])
        plsc.subcore_barrier()
        return carry

    lax.fori_loop(0, NBLK, block, 0)


_sc_call = functools.partial(
    pl.kernel,
    mesh=plsc.VectorSubcoreMesh(core_axis_name="c", subcore_axis_name="s"),
    out_type=jax.ShapeDtypeStruct((E // 32, 32), jnp.float32),
    scratch_types=(
        [pltpu.VMEM((GPB, 16), jnp.int32)] * 3
        + [pltpu.VMEM((PRB, 32), jnp.float32),
           pltpu.VMEM((FPT, N), jnp.float32),
           pltpu.VMEM((FPT, R), jnp.float32),
           pltpu.VMEM((N,), jnp.float32),
           pltpu.VMEM((N,), jnp.float32),
           pltpu.VMEM((R,), jnp.float32),
           pltpu.VMEM((H * 16,), jnp.float32),
           pltpu.VMEM((PRB // RCH, RCH), jnp.int32),
           pltpu.VMEM_SHARED((PRB, 32), jnp.float32)]
    ),
    compiler_params=pltpu.CompilerParams(
        needs_layout_passes=False, use_tc_tiling_on_sc=False),
)(_sc_body)


def kernel(edge_index, edge_type, all_embed, relation_emb,
           W1_con, b1_con, W2_con, b2_con,
           W1_src, b1_src, W2_src, b2_src,
           W1_dst, b1_dst, W2_dst, b2_dst,
           W1_edge, b1_edge, W2_edge, b2_edge):
    p, q, ssrc, sdst, sedge, w2x = _precompute(
        all_embed, relation_emb,
        W1_con, b1_con, W2_con, b2_con,
        W1_src, b1_src, W2_src, b2_src,
        W1_dst, b1_dst, W2_dst, b2_dst,
        W1_edge, b1_edge, W2_edge, b2_edge,
    )
    head = edge_index[0].astype(jnp.int32).reshape(E // 16, 16)
    tail = edge_index[1].astype(jnp.int32).reshape(E // 16, 16)
    rtype = edge_type.astype(jnp.int32).reshape(E // 16, 16)
    rows = jnp.arange(PRB, dtype=jnp.int32).reshape(PRB // RCH, RCH)
    w = _sc_call(
        head, tail, rtype, rows,
        p, ssrc.reshape(N), sdst.reshape(N), q, sedge.reshape(R),
        w2x.reshape(H * 16),
    )
    u = jax.random.uniform(jax.random.key(42), (E,), dtype=jnp.float32)
    out = _epilogue(w.reshape(E // 128, 128), u.reshape(E // 128, 128))
    return out.reshape(E)
